# Initial kernel scaffold; baseline (speedup 1.0000x reference)
import jax
import jax.numpy as jnp
from jax.experimental import pallas as pl


def kernel(query, support, query_left_connections, query_left_degrees, query_right_connections, query_right_degrees, support_left_connections, support_left_degrees, support_right_connections, support_right_degrees, symbol_emb, gcn_w_W, gcn_w_b, gcn_b, gate_temp, cg_W1, cg_b1, cg_W2, cg_b2, se_W1, se_b1, se_W2, se_b2, se_ln_g, se_ln_b, qe_W_ih, qe_W_hh, qe_b_ih, qe_b_hh):
    raise NotImplementedError("write your pallas kernel here")


# speedup vs baseline: 6.7089x; 6.7089x over previous
import functools

import jax
import jax.numpy as jnp
from jax import lax
from jax.experimental import pallas as pl
from jax.experimental.pallas import tpu as pltpu
from jax.experimental.pallas import tpu_sc as plsc

D = 64
K = 50
BQ = 4096
BS = 5
STEPS = 4

NC = 2
NS = 16
NW = NC * NS

CHUNK = 128
CPW = BQ // NW
RPW = BQ * K // NW
RCH = RPW // CHUNK

SUP_CL = 0
SUP_CR = 8
SUP_RL = 16
SUP_EL = 272
SUP_RR = 528
SUP_ER = 784
SUP_N = 1280
SPW = SUP_N // NW

QB = 128
QG = BQ // QB


def _mm_t(a, b):
    return lax.dot_general(a, b, (((1,), (1,)), ((), ())),
                           preferred_element_type=jnp.float32,
                           precision=lax.Precision.HIGHEST)


def _sc_gather_body(table, icl, icr, irl, iel, irr, ier, isup,
                    o_cl, o_cr, o_rl, o_el, o_rr, o_er, o_sup,
                    idx_q, rows_v, idx_s, rows_s, sem):
    wid = lax.axis_index("s") * NC + lax.axis_index("c")

    for ic, oc in ((icl, o_cl), (icr, o_cr)):
        pltpu.sync_copy(ic.at[pl.ds(wid * CPW, CPW)],
                        idx_q.at[pl.ds(0, CPW)])
        pltpu.async_copy(table.at[idx_q.at[pl.ds(0, CPW)]], rows_v,
                         sem).wait()
        pltpu.sync_copy(rows_v, oc.at[pl.ds(wid * CPW, CHUNK)])

    for ir, orr in ((irl, o_rl), (iel, o_el), (irr, o_rr), (ier, o_er)):
        pltpu.sync_copy(ir.at[pl.ds(wid * RPW, RPW)], idx_q)

        def body(i, _, orr=orr):
            pltpu.async_copy(
                table.at[idx_q.at[pl.ds(i * CHUNK, CHUNK)]], rows_v,
                sem).wait()
            pltpu.sync_copy(
                rows_v, orr.at[pl.ds(wid * RPW + i * CHUNK, CHUNK)])
            return 0

        lax.fori_loop(0, RCH, body, 0)

    pltpu.sync_copy(isup.at[pl.ds(wid * SPW, SPW)], idx_s)
    pltpu.async_copy(table.at[idx_s], rows_s, sem).wait()
    pltpu.sync_copy(rows_s, o_sup.at[pl.ds(wid * SPW, SPW)])


@functools.cache
def _sc_gather():
    row = lambda n: jax.ShapeDtypeStruct((n, D), jnp.float32)
    return pl.kernel(
        _sc_gather_body,
        out_type=[row(BQ), row(BQ), row(BQ * K), row(BQ * K),
                  row(BQ * K), row(BQ * K), row(SUP_N)],
        mesh=plsc.VectorSubcoreMesh(core_axis_name="c", subcore_axis_name="s"),
        scratch_types=[
            pltpu.VMEM((RPW,), jnp.int32),
            pltpu.VMEM((CHUNK, D), jnp.float32),
            pltpu.VMEM((SPW,), jnp.int32),
            pltpu.VMEM((SPW, D), jnp.float32),
            pltpu.SemaphoreType.DMA,
        ],
        compiler_params=pltpu.CompilerParams(use_tc_tiling_on_sc=False),
    )


def _neighbor(rel2, ent2, center, deg, gw, wb, gb, cw1, cb1, cw2, cb2, temp,
              nb):
    out2 = _mm_t(rel2, gw[:, :D]) + _mm_t(ent2, gw[:, D:]) + wb
    out3 = out2.reshape(nb, K, D)
    logits = jnp.sum(out3 * center[:, None, :], axis=2) * (1.0 / (D ** 0.5))
    degc = jnp.maximum(deg, 1)
    mask = lax.broadcasted_iota(jnp.int32, (nb, K), 1) < degc
    logits = jnp.where(mask, logits, -1e9)
    m = jnp.max(logits, axis=1, keepdims=True)
    e = jnp.exp(logits - m)
    att = e / jnp.sum(e, axis=1, keepdims=True)
    agg = jnp.sum(att[:, :, None] * out3, axis=1)
    gh = jnp.maximum(_mm_t(agg, cw1) + cb1, 0.0)
    gate = jax.nn.sigmoid(
        (jnp.sum(gh * cw2, axis=1, keepdims=True) + cb2) / temp)
    return gate * jnp.tanh(agg + gb) + (1.0 - gate) * jnp.tanh(center)


def _encode(x, w1, b1, w2, b2, lng, lnb):
    h = jnp.maximum(_mm_t(x, w1) + b1, 0.0)
    y = x + _mm_t(h, w2) + b2
    mu = jnp.mean(y, axis=-1, keepdims=True)
    var = jnp.mean((y - mu) ** 2, axis=-1, keepdims=True)
    return lng * (y - mu) * lax.rsqrt(var + 1e-6) + lnb


def _support_body(sup, sld, srd, gw, wb, gb, cw1, cb1, cw2, cb2, temp,
                  sw1, sb1, sw2, sb2, lng, lnb, out):
    rows = sup[...]
    args = (gw[...], wb[...], gb[...], cw1[...], cb1[...], cw2[...],
            cb2[0, 0], temp[0, 0])
    sl = _neighbor(rows[SUP_RL:SUP_RL + BS * K], rows[SUP_EL:SUP_EL + BS * K],
                   rows[SUP_CL:SUP_CL + BS], sld[...], *args, nb=BS)
    sr = _neighbor(rows[SUP_RR:SUP_RR + BS * K], rows[SUP_ER:SUP_ER + BS * K],
                   rows[SUP_CR:SUP_CR + BS], srd[...], *args, nb=BS)
    x = jnp.concatenate([sl, sr], axis=1)
    sg = _encode(x, sw1[...], sb1[...], sw2[...], sb2[...], lng[...], lnb[...])
    out[...] = jnp.mean(sg, axis=0, keepdims=True)


def _query_body(cl, cr, rl, el, rr, er, qld, qrd, sgr, gw, wb, gb,
                cw1, cb1, cw2, cb2, temp, sw1, sb1, sw2, sb2, lng, lnb,
                wih, whh, bih, bhh, out):
    args = (gw[...], wb[...], gb[...], cw1[...], cb1[...], cw2[...],
            cb2[0, 0], temp[0, 0])
    ql = _neighbor(rl[...], el[...], cl[...], qld[...], *args, nb=QB)
    qr = _neighbor(rr[...], er[...], cr[...], qrd[...], *args, nb=QB)
    x = jnp.concatenate([ql, qr], axis=1)
    qg = _encode(x, sw1[...], sb1[...], sw2[...], sb2[...], lng[...], lnb[...])

    sg = sgr[...]
    whh_v = whh[...]
    dm = 2 * D
    xw = _mm_t(qg, wih[...]) + bih[...] + bhh[...]
    r_term = _mm_t(sg, whh_v[:, dm:])

    hid = 2 * dm
    c = jnp.zeros((QB, hid), jnp.float32)
    h = qg
    for step in range(STEPS):
        if step == 0:
            gates = xw
        else:
            gates = xw + _mm_t(h, whh_v[:, :dm]) + r_term
        ig = gates[:, :hid]
        fg = gates[:, hid:2 * hid]
        gg = gates[:, 2 * hid:3 * hid]
        og = gates[:, 3 * hid:]
        c = jax.nn.sigmoid(fg) * c + jax.nn.sigmoid(ig) * jnp.tanh(gg)
        h2 = jax.nn.sigmoid(og) * jnp.tanh(c)
        h = qg + h2[:, :dm]
    out[...] = jnp.sum(h * sg, axis=1, keepdims=True)


def _full(shape):
    return pl.BlockSpec(shape, lambda i: (0,) * len(shape))


@functools.cache
def _support_call():
    v = pl.BlockSpec()
    s = pl.BlockSpec(memory_space=pltpu.SMEM)
    return pl.pallas_call(
        _support_body,
        in_specs=[v, v, v, v, v, v, v, v, v, s, s, v, v, v, v, v, v],
        out_shape=jax.ShapeDtypeStruct((1, 2 * D), jnp.float32),
    )


@functools.cache
def _query_call():
    blk = lambda shape: pl.BlockSpec(shape, lambda i: (i,) + (0,) * (len(shape) - 1))
    w = _full
    return pl.pallas_call(
        _query_body,
        grid=(QG,),
        in_specs=[
            blk((QB, D)), blk((QB, D)),
            blk((QB * K, D)), blk((QB * K, D)),
            blk((QB * K, D)), blk((QB * K, D)),
            blk((QB, 1)), blk((QB, 1)),
            w((1, 2 * D)),
            w((D, 2 * D)), w((1, D)), w((1, D)),
            w((D // 2, D)), w((1, D // 2)), w((1, D // 2)),
            pl.BlockSpec(memory_space=pltpu.SMEM),
            pl.BlockSpec(memory_space=pltpu.SMEM),
            w((4 * D, 2 * D)), w((1, 4 * D)), w((2 * D, 4 * D)), w((1, 2 * D)),
            w((1, 2 * D)), w((1, 2 * D)),
            w((16 * D, 2 * D)), w((16 * D, 4 * D)), w((1, 16 * D)),
            w((1, 16 * D)),
        ],
        out_specs=blk((QB, 1)),
        out_shape=jax.ShapeDtypeStruct((BQ, 1), jnp.float32),
    )


def kernel(query, support, query_left_connections, query_left_degrees,
           query_right_connections, query_right_degrees,
           support_left_connections, support_left_degrees,
           support_right_connections, support_right_degrees,
           symbol_emb, gcn_w_W, gcn_w_b, gcn_b, gate_temp,
           cg_W1, cg_b1, cg_W2, cg_b2,
           se_W1, se_b1, se_W2, se_b2, se_ln_g, se_ln_b,
           qe_W_ih, qe_W_hh, qe_b_ih, qe_b_hh):
    i32 = jnp.int32
    icl = query[:, 0].astype(i32)
    icr = query[:, 1].astype(i32)
    irl = query_left_connections[:, :, 0].astype(i32).reshape(-1)
    iel = query_left_connections[:, :, 1].astype(i32).reshape(-1)
    irr = query_right_connections[:, :, 0].astype(i32).reshape(-1)
    ier = query_right_connections[:, :, 1].astype(i32).reshape(-1)

    pad = lambda a, n: jnp.pad(a.astype(i32).reshape(-1), (0, n))
    isup = jnp.concatenate([
        pad(support[:, 0], 3), pad(support[:, 1], 3),
        pad(support_left_connections[:, :, 0], 6),
        pad(support_left_connections[:, :, 1], 6),
        pad(support_right_connections[:, :, 0], 6),
        pad(support_right_connections[:, :, 1], 6),
    ])
    isup = jnp.pad(isup, (0, SUP_N - isup.shape[0]))

    (cl, cr, rl, el, rr, er, sup) = _sc_gather()(
        symbol_emb, icl, icr, irl, iel, irr, ier, isup)

    r2 = lambda a, s: a.astype(jnp.float32).reshape(s)
    gw = gcn_w_W
    wb = r2(gcn_w_b, (1, D))
    gb = r2(gcn_b, (1, D))
    cb1 = r2(cg_b1, (1, D // 2))
    cb2 = r2(cg_b2, (1, 1))
    temp = r2(gate_temp, (1, 1))
    sb1 = r2(se_b1, (1, 4 * D))
    sb2 = r2(se_b2, (1, 2 * D))
    lng = r2(se_ln_g, (1, 2 * D))
    lnb = r2(se_ln_b, (1, 2 * D))
    bih = r2(qe_b_ih, (1, 16 * D))
    bhh = r2(qe_b_hh, (1, 16 * D))

    support_g = _support_call()(
        sup, support_left_degrees.astype(i32).reshape(BS, 1),
        support_right_degrees.astype(i32).reshape(BS, 1),
        gw, wb, gb, cg_W1, cb1, cg_W2, cb2, temp,
        se_W1, sb1, se_W2, sb2, lng, lnb)

    scores = _query_call()(
        cl, cr, rl, el, rr, er,
        query_left_degrees.astype(i32).reshape(BQ, 1),
        query_right_degrees.astype(i32).reshape(BQ, 1),
        support_g, gw, wb, gb, cg_W1, cb1, cg_W2, cb2, temp,
        se_W1, sb1, se_W2, sb2, lng, lnb,
        qe_W_ih, qe_W_hh, bih, bhh)
    return scores[:, 0]


# speedup vs baseline: 8.4986x; 1.2668x over previous
import functools

import jax
import jax.numpy as jnp
from jax import lax
from jax.experimental import pallas as pl
from jax.experimental.pallas import tpu as pltpu
from jax.experimental.pallas import tpu_sc as plsc

D = 64
K = 50
BQ = 4096
BS = 5
STEPS = 4

NC = 2
NS = 16
NW = NC * NS

CHUNK = 128
CN = BQ * 2 // NW
CCH = CN // CHUNK
RPW = BQ * K * 2 // NW
RCH = RPW // CHUNK

SUP_C = 0
SUP_L = 16
SUP_R = 528
SUP_N = 1280
SPW = SUP_N // NW

QB = 128
QG = BQ // QB


def _mm_t(a, b):
    return lax.dot_general(a, b, (((1,), (1,)), ((), ())),
                           preferred_element_type=jnp.float32,
                           precision=lax.Precision.HIGHEST)


def _sc_gather_body(table, icq, il, ir, isup,
                    o_qc, o_l, o_r, o_sup,
                    idx_q, rows_v, idx_s, rows_s, sem):
    wid = lax.axis_index("s") * NC + lax.axis_index("c")

    pltpu.sync_copy(icq.at[pl.ds(wid * CN, CN)], idx_q.at[pl.ds(0, CN)])
    for j in range(CCH):
        pltpu.async_copy(
            table.at[idx_q.at[pl.ds(j * CHUNK, CHUNK)]], rows_v, sem).wait()
        pltpu.sync_copy(rows_v, o_qc.at[pl.ds(wid * CN + j * CHUNK, CHUNK)])

    for iseg, oseg in ((il, o_l), (ir, o_r)):
        pltpu.sync_copy(iseg.at[pl.ds(wid * RPW, RPW)], idx_q)

        def body(i, _, oseg=oseg):
            pltpu.async_copy(
                table.at[idx_q.at[pl.ds(i * CHUNK, CHUNK)]], rows_v,
                sem).wait()
            pltpu.sync_copy(
                rows_v, oseg.at[pl.ds(wid * RPW + i * CHUNK, CHUNK)])
            return 0

        lax.fori_loop(0, RCH, body, 0)

    pltpu.sync_copy(isup.at[pl.ds(wid * SPW, SPW)], idx_s)
    pltpu.async_copy(table.at[idx_s], rows_s, sem).wait()
    pltpu.sync_copy(rows_s, o_sup.at[pl.ds(wid * SPW, SPW)])


@functools.cache
def _sc_gather():
    row = lambda n: jax.ShapeDtypeStruct((n, D), jnp.float32)
    return pl.kernel(
        _sc_gather_body,
        out_type=[row(BQ * 2), row(BQ * K * 2), row(BQ * K * 2), row(SUP_N)],
        mesh=plsc.VectorSubcoreMesh(core_axis_name="c", subcore_axis_name="s"),
        scratch_types=[
            pltpu.VMEM((RPW,), jnp.int32),
            pltpu.VMEM((CHUNK, D), jnp.float32),
            pltpu.VMEM((SPW,), jnp.int32),
            pltpu.VMEM((SPW, D), jnp.float32),
            pltpu.SemaphoreType.DMA,
        ],
        compiler_params=pltpu.CompilerParams(use_tc_tiling_on_sc=False),
    )


def _neighbor(feats2, center, deg, gw, wb, gb, cw1, cb1, cw2, cb2, temp, nb):
    out2 = _mm_t(feats2, gw) + wb
    out3 = out2.reshape(nb, K, D)
    logits = jnp.sum(out3 * center[:, None, :], axis=2) * (1.0 / (D ** 0.5))
    degc = jnp.maximum(deg, 1)
    mask = lax.broadcasted_iota(jnp.int32, (nb, K), 1) < degc
    logits = jnp.where(mask, logits, -1e9)
    m = jnp.max(logits, axis=1, keepdims=True)
    e = jnp.exp(logits - m)
    att = e / jnp.sum(e, axis=1, keepdims=True)
    agg = jnp.sum(att[:, :, None] * out3, axis=1)
    gh = jnp.maximum(_mm_t(agg, cw1) + cb1, 0.0)
    gate = jax.nn.sigmoid(
        (jnp.sum(gh * cw2, axis=1, keepdims=True) + cb2) / temp)
    return gate * jnp.tanh(agg + gb) + (1.0 - gate) * jnp.tanh(center)


def _encode(x, w1, b1, w2, b2, lng, lnb):
    h = jnp.maximum(_mm_t(x, w1) + b1, 0.0)
    y = x + _mm_t(h, w2) + b2
    mu = jnp.mean(y, axis=-1, keepdims=True)
    var = jnp.mean((y - mu) ** 2, axis=-1, keepdims=True)
    return lng * (y - mu) * lax.rsqrt(var + 1e-6) + lnb


def _support_body(sup, sld, srd, gw, wb, gb, cw1, cb1, cw2, cb2, temp,
                  sw1, sb1, sw2, sb2, lng, lnb, out):
    rows = sup[...]
    args = (gw[...], wb[...], gb[...], cw1[...], cb1[...], cw2[...],
            cb2[0, 0], temp[0, 0])
    cb = rows[SUP_C // 2:SUP_C // 2 + BS]
    fl = rows[SUP_L // 2:SUP_L // 2 + BS * K]
    fr = rows[SUP_R // 2:SUP_R // 2 + BS * K]
    sl = _neighbor(fl, cb[:, :D], sld[...], *args, nb=BS)
    sr = _neighbor(fr, cb[:, D:], srd[...], *args, nb=BS)
    x = jnp.concatenate([sl, sr], axis=1)
    sg = _encode(x, sw1[...], sb1[...], sw2[...], sb2[...], lng[...], lnb[...])
    out[...] = jnp.mean(sg, axis=0, keepdims=True)


def _query_body(qc, fl, fr, qld, qrd, sgr, gw, wb, gb,
                cw1, cb1, cw2, cb2, temp, sw1, sb1, sw2, sb2, lng, lnb,
                wih, whh, bih, bhh, out):
    args = (gw[...], wb[...], gb[...], cw1[...], cb1[...], cw2[...],
            cb2[0, 0], temp[0, 0])
    cb = qc[...]
    ql = _neighbor(fl[...], cb[:, :D], qld[...], *args, nb=QB)
    qr = _neighbor(fr[...], cb[:, D:], qrd[...], *args, nb=QB)
    x = jnp.concatenate([ql, qr], axis=1)
    qg = _encode(x, sw1[...], sb1[...], sw2[...], sb2[...], lng[...], lnb[...])

    sg = sgr[...]
    whh_v = whh[...]
    dm = 2 * D
    xw = _mm_t(qg, wih[...]) + bih[...] + bhh[...]
    r_term = _mm_t(sg, whh_v[:, dm:])

    hid = 2 * dm
    c = jnp.zeros((QB, hid), jnp.float32)
    h = qg
    for step in range(STEPS):
        if step == 0:
            gates = xw
        else:
            gates = xw + _mm_t(h, whh_v[:, :dm]) + r_term
        ig = gates[:, :hid]
        fg = gates[:, hid:2 * hid]
        gg = gates[:, 2 * hid:3 * hid]
        og = gates[:, 3 * hid:]
        c = jax.nn.sigmoid(fg) * c + jax.nn.sigmoid(ig) * jnp.tanh(gg)
        h2 = jax.nn.sigmoid(og) * jnp.tanh(c)
        h = qg + h2[:, :dm]
    out[...] = jnp.sum(h * sg, axis=1, keepdims=True)


def _full(shape):
    return pl.BlockSpec(shape, lambda i: (0,) * len(shape))


@functools.cache
def _support_call():
    v = pl.BlockSpec()
    s = pl.BlockSpec(memory_space=pltpu.SMEM)
    return pl.pallas_call(
        _support_body,
        in_specs=[v, v, v, v, v, v, v, v, v, s, s, v, v, v, v, v, v],
        out_shape=jax.ShapeDtypeStruct((1, 2 * D), jnp.float32),
    )


@functools.cache
def _query_call():
    blk = lambda shape: pl.BlockSpec(shape, lambda i: (i,) + (0,) * (len(shape) - 1))
    w = _full
    return pl.pallas_call(
        _query_body,
        grid=(QG,),
        in_specs=[
            blk((QB, 2 * D)),
            blk((QB * K, 2 * D)), blk((QB * K, 2 * D)),
            blk((QB, 1)), blk((QB, 1)),
            w((1, 2 * D)),
            w((D, 2 * D)), w((1, D)), w((1, D)),
            w((D // 2, D)), w((1, D // 2)), w((1, D // 2)),
            pl.BlockSpec(memory_space=pltpu.SMEM),
            pl.BlockSpec(memory_space=pltpu.SMEM),
            w((4 * D, 2 * D)), w((1, 4 * D)), w((2 * D, 4 * D)), w((1, 2 * D)),
            w((1, 2 * D)), w((1, 2 * D)),
            w((16 * D, 2 * D)), w((16 * D, 4 * D)), w((1, 16 * D)),
            w((1, 16 * D)),
        ],
        out_specs=blk((QB, 1)),
        out_shape=jax.ShapeDtypeStruct((BQ, 1), jnp.float32),
    )


def kernel(query, support, query_left_connections, query_left_degrees,
           query_right_connections, query_right_degrees,
           support_left_connections, support_left_degrees,
           support_right_connections, support_right_degrees,
           symbol_emb, gcn_w_W, gcn_w_b, gcn_b, gate_temp,
           cg_W1, cg_b1, cg_W2, cg_b2,
           se_W1, se_b1, se_W2, se_b2, se_ln_g, se_ln_b,
           qe_W_ih, qe_W_hh, qe_b_ih, qe_b_hh):
    i32 = jnp.int32
    icq = query.astype(i32).reshape(-1)
    il = query_left_connections.astype(i32).reshape(-1)
    ir = query_right_connections.astype(i32).reshape(-1)

    pad = lambda a, n: jnp.pad(a.astype(i32).reshape(-1), (0, n))
    isup = jnp.concatenate([
        pad(support, 6),
        pad(support_left_connections, 12),
        pad(support_right_connections, 252),
    ])

    (qc, cl, cr, sup) = _sc_gather()(symbol_emb, icq, il, ir, isup)

    r2 = lambda a, s: a.astype(jnp.float32).reshape(s)
    gw = gcn_w_W
    wb = r2(gcn_w_b, (1, D))
    gb = r2(gcn_b, (1, D))
    cb1 = r2(cg_b1, (1, D // 2))
    cb2 = r2(cg_b2, (1, 1))
    temp = r2(gate_temp, (1, 1))
    sb1 = r2(se_b1, (1, 4 * D))
    sb2 = r2(se_b2, (1, 2 * D))
    lng = r2(se_ln_g, (1, 2 * D))
    lnb = r2(se_ln_b, (1, 2 * D))
    bih = r2(qe_b_ih, (1, 16 * D))
    bhh = r2(qe_b_hh, (1, 16 * D))

    support_g = _support_call()(
        sup.reshape(SUP_N // 2, 2 * D),
        support_left_degrees.astype(i32).reshape(BS, 1),
        support_right_degrees.astype(i32).reshape(BS, 1),
        gw, wb, gb, cg_W1, cb1, cg_W2, cb2, temp,
        se_W1, sb1, se_W2, sb2, lng, lnb)

    scores = _query_call()(
        qc.reshape(BQ, 2 * D),
        cl.reshape(BQ * K, 2 * D), cr.reshape(BQ * K, 2 * D),
        query_left_degrees.astype(i32).reshape(BQ, 1),
        query_right_degrees.astype(i32).reshape(BQ, 1),
        support_g, gw, wb, gb, cg_W1, cb1, cg_W2, cb2, temp,
        se_W1, sb1, se_W2, sb2, lng, lnb,
        qe_W_ih, qe_W_hh, bih, bhh)
    return scores[:, 0]
